# baseline (speedup 1.0000x reference)
import functools

import jax
import jax.numpy as jnp
from jax import lax
from jax.experimental import pallas as pl
from jax.experimental.pallas import tpu as pltpu
from jax.experimental.pallas import tpu_sc as plsc

N_USERS = 100000
N_ITEMS = 100000
K = 64
B = 4096
L = 16
NC, NS = 2, 16
NW = NC * NS
BPW = B // NW
G = BPW // L

LANES = 128
SUBL = 8
NTILES = 782
NPAD = NTILES * LANES
TROW = SUBL * NPAD
NWORDS = (K // SUBL) * TROW

TCHUNK = 23
CW = TCHUNK * LANES
CWORDS = TCHUNK * SUBL * LANES
NCHUNK = NTILES // TCHUNK

_params = pltpu.CompilerParams(
    needs_layout_passes=False, use_tc_tiling_on_sc=False)
_mesh = plsc.VectorSubcoreMesh(core_axis_name="c", subcore_axis_name="s")


def _linearize_body(t_ref, o_ref):
    for nt in range(TCHUNK):
        o_ref[pl.ds(nt * SUBL * LANES, SUBL * LANES)] = (
            t_ref[:, pl.ds(nt * LANES, LANES)].reshape(SUBL * LANES))


def _linearize(table):
    return pl.pallas_call(
        _linearize_body,
        grid=(K // SUBL, NCHUNK),
        in_specs=[pl.BlockSpec((SUBL, CW), lambda kt, c: (kt, c))],
        out_specs=pl.BlockSpec((CWORDS,), lambda kt, c: (kt * NCHUNK + c,)),
        out_shape=jax.ShapeDtypeStruct((NWORDS,), jnp.float32),
    )(table)


def _col_offsets(idx_ref, dst, g):
    sl = pl.ds(g * L, L)
    v = idx_ref[sl]
    dst[0, sl] = ((v >> 7) << 10) | (v & (LANES - 1))


def _row_base(kk):
    return (kk >> 3) * TROW + (kk & (SUBL - 1)) * LANES


def _sc_body_a(u_hbm, pulin_hbm, puv_hbm,
               u_v, pidx, pu_vals, sem_p):
    wid = lax.axis_index("s") * NC + lax.axis_index("c")
    base = wid * BPW

    pltpu.sync_copy(u_hbm.at[pl.ds(base, BPW)], u_v)

    for g in range(G):
        _col_offsets(u_v, pidx, g)

    def build_and_fire(kk, _):
        rb = _row_base(kk)
        for g in range(G):
            sl = pl.ds(g * L, L)
            pidx[kk, sl] = pidx[0, sl] + rb
        pltpu.async_copy(pulin_hbm.at[pidx.at[kk]],
                         pu_vals.at[pl.ds(kk * BPW, BPW)], sem_p)
        return 0

    lax.fori_loop(1, K, build_and_fire, 0, unroll=False)
    build_and_fire(0, 0)

    pltpu.make_async_copy(
        pulin_hbm.at[pl.ds(0, K * BPW)], pu_vals, sem_p).wait()

    pltpu.sync_copy(pu_vals, puv_hbm.at[wid])


def _sc_body_b(i_hbm, u_hbm, bi_hbm, bu_hbm, qilin_hbm, puv_hbm, uvec_hbm,
               out_hbm,
               i_v, u_v, qidx, qi_vals, pu_vals, bu_v, bi_v, u_const,
               out_v, sem_b, sem_q, sem_s):
    wid = lax.axis_index("s") * NC + lax.axis_index("c")
    base = wid * BPW

    pltpu.sync_copy(i_hbm.at[pl.ds(base, BPW)], i_v)
    pltpu.sync_copy(u_hbm.at[pl.ds(base, BPW)], u_v)
    pltpu.sync_copy(uvec_hbm, u_const)

    cp_bu = pltpu.async_copy(bu_hbm.at[u_v], bu_v, sem_b)
    cp_bi = pltpu.async_copy(bi_hbm.at[i_v], bi_v, sem_b)
    cp_pu = pltpu.async_copy(puv_hbm.at[wid], pu_vals, sem_s)

    for g in range(G):
        _col_offsets(i_v, qidx, g)

    def build_and_fire(kk, _):
        rb = _row_base(kk)
        for g in range(G):
            sl = pl.ds(g * L, L)
            qidx[kk, sl] = qidx[0, sl] + rb
        pltpu.async_copy(qilin_hbm.at[qidx.at[kk]],
                         qi_vals.at[pl.ds(kk * BPW, BPW)], sem_q)
        return 0

    lax.fori_loop(1, K, build_and_fire, 0, unroll=False)
    build_and_fire(0, 0)

    cp_bu.wait()
    cp_bi.wait()
    cp_pu.wait()

    pltpu.make_async_copy(
        qilin_hbm.at[pl.ds(0, K * BPW)], qi_vals, sem_q).wait()

    def dot_step(kk, accs):
        out = []
        for g in range(G):
            sl = pl.ds(kk * BPW + g * L, L)
            out.append(accs[g] + pu_vals[sl] * qi_vals[sl])
        return tuple(out)

    accs = lax.fori_loop(
        0, K, dot_step,
        tuple(jnp.zeros((L,), jnp.float32) for _ in range(G)),
        unroll=False)

    uc = u_const[...]
    for g in range(G):
        sl = pl.ds(g * L, L)
        out_v[sl] = uc + bu_v[sl] + bi_v[sl] + accs[g]
    pltpu.sync_copy(out_v, out_hbm.at[pl.ds(base, BPW)])


@jax.jit
def _run(u, i, bi, bu, qi, puT, u_vec):
    pu_lin = _linearize(puT)
    qi_lin = _linearize(qi)

    ka = functools.partial(
        pl.kernel,
        mesh=_mesh,
        compiler_params=_params,
        out_type=jax.ShapeDtypeStruct((NW, K * BPW), jnp.float32),
        scratch_types=[
            pltpu.VMEM((BPW,), jnp.int32),
            pltpu.VMEM((K, BPW), jnp.int32),
            pltpu.VMEM((K * BPW,), jnp.float32),
            pltpu.SemaphoreType.DMA,
        ],
    )(_sc_body_a)
    puv = ka(u, pu_lin)

    kb = functools.partial(
        pl.kernel,
        mesh=_mesh,
        compiler_params=_params,
        out_type=jax.ShapeDtypeStruct((B,), jnp.float32),
        scratch_types=[
            pltpu.VMEM((BPW,), jnp.int32),
            pltpu.VMEM((BPW,), jnp.int32),
            pltpu.VMEM((K, BPW), jnp.int32),
            pltpu.VMEM((K * BPW,), jnp.float32),
            pltpu.VMEM((K * BPW,), jnp.float32),
            pltpu.VMEM((BPW,), jnp.float32),
            pltpu.VMEM((BPW,), jnp.float32),
            pltpu.VMEM((L,), jnp.float32),
            pltpu.VMEM((BPW,), jnp.float32),
            pltpu.SemaphoreType.DMA,
            pltpu.SemaphoreType.DMA,
            pltpu.SemaphoreType.DMA,
        ],
    )(_sc_body_b)
    return kb(i, u, bi, bu, qi_lin, puv, u_vec)


def kernel(u, i, bi, bu, qi, pu, U):
    u_vec = jnp.full((L,), U, jnp.float32)
    return _run(u, i, bi, bu, qi, pu.T, u_vec)


# speedup vs baseline: 1.0029x; 1.0029x over previous
import functools

import jax
import jax.numpy as jnp
from jax import lax
from jax.experimental import pallas as pl
from jax.experimental.pallas import tpu as pltpu
from jax.experimental.pallas import tpu_sc as plsc

N_USERS = 100000
N_ITEMS = 100000
K = 64
B = 4096
L = 16
NC, NS = 2, 16
NW = NC * NS
BPW = B // NW
G = BPW // L

LANES = 128
SUBL = 8
NTILES = 782
NPAD = NTILES * LANES
TROW = SUBL * NPAD
NWORDS = (K // SUBL) * TROW

TCHUNK = 23
CW = TCHUNK * LANES
CWORDS = TCHUNK * SUBL * LANES
NCHUNK = NTILES // TCHUNK

_params = pltpu.CompilerParams(
    needs_layout_passes=False, use_tc_tiling_on_sc=False)
_mesh = plsc.VectorSubcoreMesh(core_axis_name="c", subcore_axis_name="s")


def _linearize_body(t_ref, o_ref):
    for nt in range(TCHUNK):
        o_ref[pl.ds(nt * SUBL, SUBL), :] = t_ref[:, pl.ds(nt * LANES, LANES)]


def _linearize(table):
    out2d = pl.pallas_call(
        _linearize_body,
        grid=(K // SUBL, NCHUNK),
        in_specs=[pl.BlockSpec((SUBL, CW), lambda kt, c: (kt, c))],
        out_specs=pl.BlockSpec((TCHUNK * SUBL, LANES),
                               lambda kt, c: (kt * NCHUNK + c, 0)),
        out_shape=jax.ShapeDtypeStruct((NWORDS // LANES, LANES), jnp.float32),
    )(table)
    return out2d.reshape(-1)


def _col_offsets(idx_ref, dst, g):
    sl = pl.ds(g * L, L)
    v = idx_ref[sl]
    dst[0, sl] = ((v >> 7) << 10) | (v & (LANES - 1))


def _row_base(kk):
    return (kk >> 3) * TROW + (kk & (SUBL - 1)) * LANES


def _sc_body_a(u_hbm, pulin_hbm, puv_hbm,
               u_v, pidx, pu_vals, sem_p):
    wid = lax.axis_index("s") * NC + lax.axis_index("c")
    base = wid * BPW

    pltpu.sync_copy(u_hbm.at[pl.ds(base, BPW)], u_v)

    for g in range(G):
        _col_offsets(u_v, pidx, g)

    def build_and_fire(kk, _):
        rb = _row_base(kk)
        for g in range(G):
            sl = pl.ds(g * L, L)
            pidx[kk, sl] = pidx[0, sl] + rb
        pltpu.async_copy(pulin_hbm.at[pidx.at[kk]],
                         pu_vals.at[pl.ds(kk * BPW, BPW)], sem_p)
        return 0

    lax.fori_loop(1, K, build_and_fire, 0, unroll=False)
    build_and_fire(0, 0)

    pltpu.make_async_copy(
        pulin_hbm.at[pl.ds(0, K * BPW)], pu_vals, sem_p).wait()

    pltpu.sync_copy(pu_vals, puv_hbm.at[wid])


def _sc_body_b(i_hbm, u_hbm, bi_hbm, bu_hbm, qilin_hbm, puv_hbm, uvec_hbm,
               out_hbm,
               i_v, u_v, qidx, qi_vals, pu_vals, bu_v, bi_v, u_const,
               out_v, sem_b, sem_q, sem_s):
    wid = lax.axis_index("s") * NC + lax.axis_index("c")
    base = wid * BPW

    pltpu.sync_copy(i_hbm.at[pl.ds(base, BPW)], i_v)
    pltpu.sync_copy(u_hbm.at[pl.ds(base, BPW)], u_v)
    pltpu.sync_copy(uvec_hbm, u_const)

    cp_bu = pltpu.async_copy(bu_hbm.at[u_v], bu_v, sem_b)
    cp_bi = pltpu.async_copy(bi_hbm.at[i_v], bi_v, sem_b)
    cp_pu = pltpu.async_copy(puv_hbm.at[wid], pu_vals, sem_s)

    for g in range(G):
        _col_offsets(i_v, qidx, g)

    def build_and_fire(kk, _):
        rb = _row_base(kk)
        for g in range(G):
            sl = pl.ds(g * L, L)
            qidx[kk, sl] = qidx[0, sl] + rb
        pltpu.async_copy(qilin_hbm.at[qidx.at[kk]],
                         qi_vals.at[pl.ds(kk * BPW, BPW)], sem_q)
        return 0

    lax.fori_loop(1, K, build_and_fire, 0, unroll=False)
    build_and_fire(0, 0)

    cp_bu.wait()
    cp_bi.wait()
    cp_pu.wait()

    pltpu.make_async_copy(
        qilin_hbm.at[pl.ds(0, K * BPW)], qi_vals, sem_q).wait()

    def dot_step(kk, accs):
        out = []
        for g in range(G):
            sl = pl.ds(kk * BPW + g * L, L)
            out.append(accs[g] + pu_vals[sl] * qi_vals[sl])
        return tuple(out)

    accs = lax.fori_loop(
        0, K, dot_step,
        tuple(jnp.zeros((L,), jnp.float32) for _ in range(G)),
        unroll=False)

    uc = u_const[...]
    for g in range(G):
        sl = pl.ds(g * L, L)
        out_v[sl] = uc + bu_v[sl] + bi_v[sl] + accs[g]
    pltpu.sync_copy(out_v, out_hbm.at[pl.ds(base, BPW)])


@jax.jit
def _run(u, i, bi, bu, qi, puT, u_vec):
    pu_lin = _linearize(puT)
    qi_lin = _linearize(qi)

    ka = functools.partial(
        pl.kernel,
        mesh=_mesh,
        compiler_params=_params,
        out_type=jax.ShapeDtypeStruct((NW, K * BPW), jnp.float32),
        scratch_types=[
            pltpu.VMEM((BPW,), jnp.int32),
            pltpu.VMEM((K, BPW), jnp.int32),
            pltpu.VMEM((K * BPW,), jnp.float32),
            pltpu.SemaphoreType.DMA,
        ],
    )(_sc_body_a)
    puv = ka(u, pu_lin)

    kb = functools.partial(
        pl.kernel,
        mesh=_mesh,
        compiler_params=_params,
        out_type=jax.ShapeDtypeStruct((B,), jnp.float32),
        scratch_types=[
            pltpu.VMEM((BPW,), jnp.int32),
            pltpu.VMEM((BPW,), jnp.int32),
            pltpu.VMEM((K, BPW), jnp.int32),
            pltpu.VMEM((K * BPW,), jnp.float32),
            pltpu.VMEM((K * BPW,), jnp.float32),
            pltpu.VMEM((BPW,), jnp.float32),
            pltpu.VMEM((BPW,), jnp.float32),
            pltpu.VMEM((L,), jnp.float32),
            pltpu.VMEM((BPW,), jnp.float32),
            pltpu.SemaphoreType.DMA,
            pltpu.SemaphoreType.DMA,
            pltpu.SemaphoreType.DMA,
        ],
    )(_sc_body_b)
    return kb(i, u, bi, bu, qi_lin, puv, u_vec)


def kernel(u, i, bi, bu, qi, pu, U):
    u_vec = jnp.full((L,), U, jnp.float32)
    return _run(u, i, bi, bu, qi, pu.T, u_vec)


# speedup vs baseline: 4.2542x; 4.2418x over previous
import functools

import jax
import jax.numpy as jnp
from jax import lax
from jax.experimental import pallas as pl
from jax.experimental.pallas import tpu as pltpu
from jax.experimental.pallas import tpu_sc as plsc

N_USERS = 100000
N_ITEMS = 100000
K = 64
B = 4096
L = 16
NC, NS = 2, 16
NW = NC * NS
BPW = B // NW
G = BPW // L

LANES = 128
SUBL = 8
NTILES = 782
NPAD = NTILES * LANES
TROW = SUBL * NPAD
NWORDS = (K // SUBL) * TROW

TCHUNK = 391
CW = TCHUNK * LANES
CWORDS = TCHUNK * SUBL * LANES
NCHUNK = NTILES // TCHUNK

_params = pltpu.CompilerParams(
    needs_layout_passes=False, use_tc_tiling_on_sc=False)
_mesh = plsc.VectorSubcoreMesh(core_axis_name="c", subcore_axis_name="s")


def _linearize_body(t_ref, o_ref):
    for nt in range(TCHUNK):
        o_ref[pl.ds(nt * SUBL, SUBL), :] = t_ref[:, pl.ds(nt * LANES, LANES)]


def _linearize(table):
    out2d = pl.pallas_call(
        _linearize_body,
        grid=(K // SUBL, NCHUNK),
        in_specs=[pl.BlockSpec((SUBL, CW), lambda kt, c: (kt, c))],
        out_specs=pl.BlockSpec((TCHUNK * SUBL, LANES),
                               lambda kt, c: (kt * NCHUNK + c, 0)),
        out_shape=jax.ShapeDtypeStruct((NWORDS // LANES, LANES), jnp.float32),
    )(table)
    return out2d.reshape(-1)


def _col_offsets(idx_ref, dst, g):
    sl = pl.ds(g * L, L)
    v = idx_ref[sl]
    dst[0, sl] = ((v >> 7) << 10) | (v & (LANES - 1))


def _row_base(kk):
    return (kk >> 3) * TROW + (kk & (SUBL - 1)) * LANES


def _sc_body_a(u_hbm, pulin_hbm, puv_hbm,
               u_v, pidx, pu_vals, sem_p):
    wid = lax.axis_index("s") * NC + lax.axis_index("c")
    base = wid * BPW

    pltpu.sync_copy(u_hbm.at[pl.ds(base, BPW)], u_v)

    for g in range(G):
        _col_offsets(u_v, pidx, g)

    def build_and_fire(kk, _):
        rb = _row_base(kk)
        for g in range(G):
            sl = pl.ds(g * L, L)
            pidx[kk, sl] = pidx[0, sl] + rb
        pltpu.async_copy(pulin_hbm.at[pidx.at[kk]],
                         pu_vals.at[pl.ds(kk * BPW, BPW)], sem_p)
        return 0

    lax.fori_loop(1, K, build_and_fire, 0, unroll=False)
    build_and_fire(0, 0)

    pltpu.make_async_copy(
        pulin_hbm.at[pl.ds(0, K * BPW)], pu_vals, sem_p).wait()

    pltpu.sync_copy(pu_vals, puv_hbm.at[wid])


def _sc_body_b(i_hbm, u_hbm, bi_hbm, bu_hbm, qilin_hbm, puv_hbm, uvec_hbm,
               out_hbm,
               i_v, u_v, qidx, qi_vals, pu_vals, bu_v, bi_v, u_const,
               out_v, sem_b, sem_q, sem_s):
    wid = lax.axis_index("s") * NC + lax.axis_index("c")
    base = wid * BPW

    pltpu.sync_copy(i_hbm.at[pl.ds(base, BPW)], i_v)
    pltpu.sync_copy(u_hbm.at[pl.ds(base, BPW)], u_v)
    pltpu.sync_copy(uvec_hbm, u_const)

    cp_bu = pltpu.async_copy(bu_hbm.at[u_v], bu_v, sem_b)
    cp_bi = pltpu.async_copy(bi_hbm.at[i_v], bi_v, sem_b)
    cp_pu = pltpu.async_copy(puv_hbm.at[wid], pu_vals, sem_s)

    for g in range(G):
        _col_offsets(i_v, qidx, g)

    def build_and_fire(kk, _):
        rb = _row_base(kk)
        for g in range(G):
            sl = pl.ds(g * L, L)
            qidx[kk, sl] = qidx[0, sl] + rb
        pltpu.async_copy(qilin_hbm.at[qidx.at[kk]],
                         qi_vals.at[pl.ds(kk * BPW, BPW)], sem_q)
        return 0

    lax.fori_loop(1, K, build_and_fire, 0, unroll=False)
    build_and_fire(0, 0)

    cp_bu.wait()
    cp_bi.wait()
    cp_pu.wait()

    pltpu.make_async_copy(
        qilin_hbm.at[pl.ds(0, K * BPW)], qi_vals, sem_q).wait()

    def dot_step(kk, accs):
        out = []
        for g in range(G):
            sl = pl.ds(kk * BPW + g * L, L)
            out.append(accs[g] + pu_vals[sl] * qi_vals[sl])
        return tuple(out)

    accs = lax.fori_loop(
        0, K, dot_step,
        tuple(jnp.zeros((L,), jnp.float32) for _ in range(G)),
        unroll=False)

    uc = u_const[...]
    for g in range(G):
        sl = pl.ds(g * L, L)
        out_v[sl] = uc + bu_v[sl] + bi_v[sl] + accs[g]
    pltpu.sync_copy(out_v, out_hbm.at[pl.ds(base, BPW)])


@jax.jit
def _run(u, i, bi, bu, qi, puT, u_vec):
    pu_lin = _linearize(puT)
    qi_lin = _linearize(qi)

    ka = functools.partial(
        pl.kernel,
        mesh=_mesh,
        compiler_params=_params,
        out_type=jax.ShapeDtypeStruct((NW, K * BPW), jnp.float32),
        scratch_types=[
            pltpu.VMEM((BPW,), jnp.int32),
            pltpu.VMEM((K, BPW), jnp.int32),
            pltpu.VMEM((K * BPW,), jnp.float32),
            pltpu.SemaphoreType.DMA,
        ],
    )(_sc_body_a)
    puv = ka(u, pu_lin)

    kb = functools.partial(
        pl.kernel,
        mesh=_mesh,
        compiler_params=_params,
        out_type=jax.ShapeDtypeStruct((B,), jnp.float32),
        scratch_types=[
            pltpu.VMEM((BPW,), jnp.int32),
            pltpu.VMEM((BPW,), jnp.int32),
            pltpu.VMEM((K, BPW), jnp.int32),
            pltpu.VMEM((K * BPW,), jnp.float32),
            pltpu.VMEM((K * BPW,), jnp.float32),
            pltpu.VMEM((BPW,), jnp.float32),
            pltpu.VMEM((BPW,), jnp.float32),
            pltpu.VMEM((L,), jnp.float32),
            pltpu.VMEM((BPW,), jnp.float32),
            pltpu.SemaphoreType.DMA,
            pltpu.SemaphoreType.DMA,
            pltpu.SemaphoreType.DMA,
        ],
    )(_sc_body_b)
    return kb(i, u, bi, bu, qi_lin, puv, u_vec)


def kernel(u, i, bi, bu, qi, pu, U):
    u_vec = jnp.full((L,), U, jnp.float32)
    return _run(u, i, bi, bu, qi, pu.T, u_vec)


# speedup vs baseline: 4.3805x; 1.0297x over previous
import functools

import jax
import jax.numpy as jnp
from jax import lax
from jax.experimental import pallas as pl
from jax.experimental.pallas import tpu as pltpu
from jax.experimental.pallas import tpu_sc as plsc

N_USERS = 100000
N_ITEMS = 100000
K = 64
B = 4096
L = 16
NC, NS = 2, 16
NW = NC * NS
BPW = B // NW
G = BPW // L

LANES = 128
SUBL = 8
NTILES = 782
NPAD = NTILES * LANES
TROW = SUBL * NPAD
NWORDS = (K // SUBL) * TROW

TCHUNK = 782
CW = TCHUNK * LANES
CWORDS = TCHUNK * SUBL * LANES
NCHUNK = NTILES // TCHUNK

_params = pltpu.CompilerParams(
    needs_layout_passes=False, use_tc_tiling_on_sc=False)
_mesh = plsc.VectorSubcoreMesh(core_axis_name="c", subcore_axis_name="s")


def _linearize_body(t_ref, o_ref):
    for nt in range(TCHUNK):
        o_ref[pl.ds(nt * SUBL, SUBL), :] = t_ref[:, pl.ds(nt * LANES, LANES)]


def _linearize(table):
    out2d = pl.pallas_call(
        _linearize_body,
        grid=(K // SUBL, NCHUNK),
        in_specs=[pl.BlockSpec((SUBL, CW), lambda kt, c: (kt, c))],
        out_specs=pl.BlockSpec((TCHUNK * SUBL, LANES),
                               lambda kt, c: (kt * NCHUNK + c, 0)),
        out_shape=jax.ShapeDtypeStruct((NWORDS // LANES, LANES), jnp.float32),
    )(table)
    return out2d.reshape(-1)


def _col_offsets(idx_ref, dst, g):
    sl = pl.ds(g * L, L)
    v = idx_ref[sl]
    dst[0, sl] = ((v >> 7) << 10) | (v & (LANES - 1))


def _row_base(kk):
    return (kk >> 3) * TROW + (kk & (SUBL - 1)) * LANES


def _sc_body_a(u_hbm, pulin_hbm, puv_hbm,
               u_v, pidx, pu_vals, sem_p):
    wid = lax.axis_index("s") * NC + lax.axis_index("c")
    base = wid * BPW

    pltpu.sync_copy(u_hbm.at[pl.ds(base, BPW)], u_v)

    for g in range(G):
        _col_offsets(u_v, pidx, g)

    def build_and_fire(kk, _):
        rb = _row_base(kk)
        for g in range(G):
            sl = pl.ds(g * L, L)
            pidx[kk, sl] = pidx[0, sl] + rb
        pltpu.async_copy(pulin_hbm.at[pidx.at[kk]],
                         pu_vals.at[pl.ds(kk * BPW, BPW)], sem_p)
        return 0

    lax.fori_loop(1, K, build_and_fire, 0, unroll=False)
    build_and_fire(0, 0)

    pltpu.make_async_copy(
        pulin_hbm.at[pl.ds(0, K * BPW)], pu_vals, sem_p).wait()

    pltpu.sync_copy(pu_vals, puv_hbm.at[wid])


def _sc_body_b(i_hbm, u_hbm, bi_hbm, bu_hbm, qilin_hbm, puv_hbm, uvec_hbm,
               out_hbm,
               i_v, u_v, qidx, qi_vals, pu_vals, bu_v, bi_v, u_const,
               out_v, sem_b, sem_q, sem_s):
    wid = lax.axis_index("s") * NC + lax.axis_index("c")
    base = wid * BPW

    pltpu.sync_copy(i_hbm.at[pl.ds(base, BPW)], i_v)
    pltpu.sync_copy(u_hbm.at[pl.ds(base, BPW)], u_v)
    pltpu.sync_copy(uvec_hbm, u_const)

    cp_bu = pltpu.async_copy(bu_hbm.at[u_v], bu_v, sem_b)
    cp_bi = pltpu.async_copy(bi_hbm.at[i_v], bi_v, sem_b)
    cp_pu = pltpu.async_copy(puv_hbm.at[wid], pu_vals, sem_s)

    for g in range(G):
        _col_offsets(i_v, qidx, g)

    def build_and_fire(kk, _):
        rb = _row_base(kk)
        for g in range(G):
            sl = pl.ds(g * L, L)
            qidx[kk, sl] = qidx[0, sl] + rb
        pltpu.async_copy(qilin_hbm.at[qidx.at[kk]],
                         qi_vals.at[pl.ds(kk * BPW, BPW)], sem_q)
        return 0

    lax.fori_loop(1, K, build_and_fire, 0, unroll=False)
    build_and_fire(0, 0)

    cp_bu.wait()
    cp_bi.wait()
    cp_pu.wait()

    pltpu.make_async_copy(
        qilin_hbm.at[pl.ds(0, K * BPW)], qi_vals, sem_q).wait()

    def dot_step(kk, accs):
        out = []
        for g in range(G):
            sl = pl.ds(kk * BPW + g * L, L)
            out.append(accs[g] + pu_vals[sl] * qi_vals[sl])
        return tuple(out)

    accs = lax.fori_loop(
        0, K, dot_step,
        tuple(jnp.zeros((L,), jnp.float32) for _ in range(G)),
        unroll=False)

    uc = u_const[...]
    for g in range(G):
        sl = pl.ds(g * L, L)
        out_v[sl] = uc + bu_v[sl] + bi_v[sl] + accs[g]
    pltpu.sync_copy(out_v, out_hbm.at[pl.ds(base, BPW)])


@jax.jit
def _run(u, i, bi, bu, qi, puT, u_vec):
    pu_lin = _linearize(puT)
    qi_lin = _linearize(qi)

    ka = functools.partial(
        pl.kernel,
        mesh=_mesh,
        compiler_params=_params,
        out_type=jax.ShapeDtypeStruct((NW, K * BPW), jnp.float32),
        scratch_types=[
            pltpu.VMEM((BPW,), jnp.int32),
            pltpu.VMEM((K, BPW), jnp.int32),
            pltpu.VMEM((K * BPW,), jnp.float32),
            pltpu.SemaphoreType.DMA,
        ],
    )(_sc_body_a)
    puv = ka(u, pu_lin)

    kb = functools.partial(
        pl.kernel,
        mesh=_mesh,
        compiler_params=_params,
        out_type=jax.ShapeDtypeStruct((B,), jnp.float32),
        scratch_types=[
            pltpu.VMEM((BPW,), jnp.int32),
            pltpu.VMEM((BPW,), jnp.int32),
            pltpu.VMEM((K, BPW), jnp.int32),
            pltpu.VMEM((K * BPW,), jnp.float32),
            pltpu.VMEM((K * BPW,), jnp.float32),
            pltpu.VMEM((BPW,), jnp.float32),
            pltpu.VMEM((BPW,), jnp.float32),
            pltpu.VMEM((L,), jnp.float32),
            pltpu.VMEM((BPW,), jnp.float32),
            pltpu.SemaphoreType.DMA,
            pltpu.SemaphoreType.DMA,
            pltpu.SemaphoreType.DMA,
        ],
    )(_sc_body_b)
    return kb(i, u, bi, bu, qi_lin, puv, u_vec)


def kernel(u, i, bi, bu, qi, pu, U):
    u_vec = jnp.full((L,), U, jnp.float32)
    return _run(u, i, bi, bu, qi, pu.T, u_vec)
